# Initial kernel scaffold; baseline (speedup 1.0000x reference)
import jax
import jax.numpy as jnp
from jax.experimental import pallas as pl


def kernel(x_user, x_item, ei_clicks, ei_rev, emb_user, emb_item, pre_W_user, pre_b_user, pre_W_item, pre_b_item, lin_l_W_clicks, lin_l_b_clicks, lin_r_W_clicks, lin_l_W_rev, lin_l_b_rev, lin_r_W_rev, proj_W_user, proj_b_user, proj_W_item, proj_b_item, head_W_user, head_b_user, head_W_item, head_b_item, bil_W_clicks, bil_b_clicks, bil_W_rev, bil_b_rev):
    raise NotImplementedError("write your pallas kernel here")


# speedup vs baseline: 6.1268x; 6.1268x over previous
import functools

import jax
import jax.numpy as jnp
from jax import lax
from jax.experimental import pallas as pl
from jax.experimental.pallas import tpu as pltpu
from jax.experimental.pallas import tpu_sc as plsc

N = 50000
E = 800000
FDIM = 8
BINS = 64
EMB = 16
HID = 64
ZD = 32
HALF = HID // 2

NC = 2
NS = 16

BN = 2000
CH = 1000
SCH = 400
SEG_PER_SUB = E // NS
SEG_CHUNKS = SEG_PER_SUB // SCH
WCH = 400
WTOT = N // WCH
WROUNDS = (WTOT + NS - 1) // NS
BIL_PER_SUB = E // (NC * NS)
BIL_CHUNKS = BIL_PER_SUB // CH
FULL_GROUPS = CH // 16
TAIL = CH - FULL_GROUPS * 16


def _encode_body(x_ref, emb_ref, preW_ref, preb_ref, lo_ref, hi_ref):
    x = x_ref[...]
    acc = jnp.broadcast_to(preb_ref[...][None, :], (BN, HID))
    iota_b = lax.broadcasted_iota(jnp.int32, (1, BINS), 1)
    for f in range(FDIM):
        tf = jnp.dot(emb_ref[f], preW_ref[pl.ds(f * EMB, EMB), :],
                     preferred_element_type=jnp.float32)
        oh = (x[:, f:f + 1] == iota_b).astype(jnp.float32)
        acc = acc + jnp.dot(oh, tf, preferred_element_type=jnp.float32)
    h = jnp.maximum(acc, 0.0)
    lo_ref[...] = h[:, :HALF]
    hi_ref[...] = h[:, HALF:]


def _tc_encode(x, emb, preW, preb):
    grid = N // BN
    return pl.pallas_call(
        _encode_body,
        grid=(grid,),
        in_specs=[
            pl.BlockSpec((BN, FDIM), lambda i: (i, 0)),
            pl.BlockSpec((FDIM, BINS, EMB), lambda i: (0, 0, 0)),
            pl.BlockSpec((FDIM * EMB, HID), lambda i: (0, 0)),
            pl.BlockSpec((HID,), lambda i: (0,)),
        ],
        out_specs=[
            pl.BlockSpec((BN, HALF), lambda i: (i, 0)),
            pl.BlockSpec((BN, HALF), lambda i: (i, 0)),
        ],
        out_shape=[
            jax.ShapeDtypeStruct((N, HALF), jnp.float32),
            jax.ShapeDtypeStruct((N, HALF), jnp.float32),
        ],
    )(x, emb, preW, preb)


def _seg_body(h0ulo, h0uhi, h0ilo, h0ihi, srcC, dstC, srcR, dstR,
              zeros_h, ones_h,
              sumc_lo, sumc_hi, sumr_lo, sumr_hi, cntc, cntr,
              acc, sidx, didx, rows, gsem):
    c = lax.axis_index("c")
    s = lax.axis_index("s")

    def each_chunk(fn):
        def t_body(t, carry):
            k = s + NS * t

            @pl.when(k < WTOT)
            def _():
                fn(k * WCH)
            return carry
        lax.fori_loop(0, WROUNDS, t_body, 0)

    def zero_acc():
        pltpu.sync_copy(zeros_h, rows)
        each_chunk(lambda r0: pltpu.sync_copy(rows, acc.at[pl.ds(r0, WCH)]))

    def seg_phase(table, src, dst):
        def ch_body(i, carry):
            eb = s * SEG_PER_SUB + i * SCH
            pltpu.sync_copy(src.at[pl.ds(eb, SCH)], sidx)
            pltpu.sync_copy(dst.at[pl.ds(eb, SCH)], didx)
            pltpu.async_copy(table.at[sidx], rows, gsem).wait()
            pltpu.sync_copy(rows, acc.at[didx], add=True)
            return carry
        lax.fori_loop(0, SEG_CHUNKS, ch_body, 0)

    def cnt_phase(dst):
        pltpu.sync_copy(ones_h, rows)

        def ch_body(i, carry):
            eb = s * SEG_PER_SUB + i * SCH
            pltpu.sync_copy(dst.at[pl.ds(eb, SCH)], didx)
            pltpu.sync_copy(rows, acc.at[didx], add=True)
            return carry
        lax.fori_loop(0, SEG_CHUNKS, ch_body, 0)

    def writeout(out):
        each_chunk(
            lambda r0: pltpu.sync_copy(acc.at[pl.ds(r0, WCH)],
                                       out.at[pl.ds(r0, WCH)]))

    zero_acc()
    plsc.subcore_barrier()

    @pl.when(c == 0)
    def _():
        seg_phase(h0ulo, srcC, dstC)

    @pl.when(c == 1)
    def _():
        seg_phase(h0uhi, srcC, dstC)

    plsc.subcore_barrier()

    @pl.when(c == 0)
    def _():
        writeout(sumc_lo)

    @pl.when(c == 1)
    def _():
        writeout(sumc_hi)

    plsc.subcore_barrier()

    zero_acc()
    plsc.subcore_barrier()

    @pl.when(c == 0)
    def _():
        seg_phase(h0ilo, srcR, dstR)

    @pl.when(c == 1)
    def _():
        seg_phase(h0ihi, srcR, dstR)

    plsc.subcore_barrier()

    @pl.when(c == 0)
    def _():
        writeout(sumr_lo)

    @pl.when(c == 1)
    def _():
        writeout(sumr_hi)

    plsc.subcore_barrier()

    zero_acc()
    plsc.subcore_barrier()

    @pl.when(c == 0)
    def _():
        cnt_phase(dstC)

    @pl.when(c == 1)
    def _():
        cnt_phase(dstR)

    plsc.subcore_barrier()

    @pl.when(c == 0)
    def _():
        writeout(cntc)

    @pl.when(c == 1)
    def _():
        writeout(cntr)


def _sc_segments(h0ulo, h0uhi, h0ilo, h0ihi, srcC, dstC, srcR, dstR):
    mesh = plsc.VectorSubcoreMesh(core_axis_name="c", subcore_axis_name="s", num_cores=NC, num_subcores=NS)
    zeros_h = jnp.zeros((SCH, HALF), jnp.float32)
    ones_h = jnp.ones((SCH, HALF), jnp.float32)
    f = pl.kernel(
        _seg_body,
        compiler_params=pltpu.CompilerParams(use_tc_tiling_on_sc=False, needs_layout_passes=False),
        out_type=[jax.ShapeDtypeStruct((N, HALF), jnp.float32)] * 4
        + [jax.ShapeDtypeStruct((N, HALF), jnp.float32)] * 2,
        mesh=mesh,
        scratch_types=[
            pltpu.VMEM_SHARED((N, HALF), jnp.float32),
            pltpu.VMEM((SCH,), jnp.int32),
            pltpu.VMEM((SCH,), jnp.int32),
            pltpu.VMEM((SCH, HALF), jnp.float32),
            pltpu.SemaphoreType.DMA,
        ],
    )
    return f(h0ulo, h0uhi, h0ilo, h0ihi, srcC, dstC, srcR, dstR, zeros_h, ones_h)


def _dense_body(slo_ref, shi_ref, cnt_ref, h0lo_ref, h0hi_ref,
                lW_ref, lb_ref, rW_ref, pW_ref, pb_ref, bW_ref,
                z_ref, a_ref):
    inv = 1.0 / jnp.maximum(cnt_ref[:, 0:1], 1.0)
    agg_lo = slo_ref[...] * inv
    agg_hi = shi_ref[...] * inv
    h = (jnp.dot(agg_lo, lW_ref[:HALF, :], preferred_element_type=jnp.float32)
         + jnp.dot(agg_hi, lW_ref[HALF:, :], preferred_element_type=jnp.float32)
         + lb_ref[...][None, :]
         + jnp.dot(h0lo_ref[...], rW_ref[:HALF, :], preferred_element_type=jnp.float32)
         + jnp.dot(h0hi_ref[...], rW_ref[HALF:, :], preferred_element_type=jnp.float32))
    z = jnp.maximum(jnp.dot(h, pW_ref[...], preferred_element_type=jnp.float32)
                    + pb_ref[...][None, :], 0.0)
    z_ref[...] = z
    a_ref[...] = jnp.dot(z, bW_ref[...], preferred_element_type=jnp.float32)


def _tc_dense(slo, shi, cnt, h0lo, h0hi, lW, lb, rW, pW, pb, bW):
    grid = N // BN
    full = lambda shape: pl.BlockSpec(shape, lambda i: tuple(0 for _ in shape))
    blk = lambda w: pl.BlockSpec((BN, w), lambda i: (i, 0))
    return pl.pallas_call(
        _dense_body,
        grid=(grid,),
        in_specs=[
            blk(HALF), blk(HALF), blk(HALF), blk(HALF), blk(HALF),
            full((HID, HID)), full((HID,)), full((HID, HID)),
            full((HID, ZD)), full((ZD,)), full((ZD, ZD)),
        ],
        out_specs=[blk(ZD), blk(ZD)],
        out_shape=[
            jax.ShapeDtypeStruct((N, ZD), jnp.float32),
            jax.ShapeDtypeStruct((N, ZD), jnp.float32),
        ],
    )(slo, shi, cnt, h0lo, h0hi, lW, lb, rW, pW, pb, bW)


def _head_body(z_ref, hW_ref, hb_ref, fl_ref):
    fl_ref[...] = (jnp.dot(z_ref[...], hW_ref[...],
                           preferred_element_type=jnp.float32)
                   + hb_ref[...][None, :])


def _tc_head(z, hW, hb):
    grid = N // BN
    return pl.pallas_call(
        _head_body,
        grid=(grid,),
        in_specs=[
            pl.BlockSpec((BN, ZD), lambda i: (i, 0)),
            pl.BlockSpec((ZD, FDIM * BINS), lambda i: (0, 0)),
            pl.BlockSpec((FDIM * BINS,), lambda i: (0,)),
        ],
        out_specs=pl.BlockSpec((BN, FDIM * BINS), lambda i: (i, 0)),
        out_shape=jax.ShapeDtypeStruct((N, FDIM * BINS), jnp.float32),
    )(z, hW, hb)


def _bil_body(au, zi_t, ai, zu_t, srcC, dstC, srcR, dstR, biasC, biasR,
              outC, outR,
              sidx, didx, arows, brows, outv, bias_v, gsem):
    c = lax.axis_index("c")
    s = lax.axis_index("s")
    wid = s * NC + c
    iota16 = lax.iota(jnp.int32, 16)

    def relation(atab, btab, src, dst, out_hbm, bias_h):
        pltpu.sync_copy(bias_h, bias_v)
        bvec = bias_v[...]

        def ch_body(i, carry):
            eb = wid * BIL_PER_SUB + i * CH
            pltpu.sync_copy(src.at[pl.ds(eb, CH)], sidx)
            pltpu.sync_copy(dst.at[pl.ds(eb, CH)], didx)
            pltpu.async_copy(atab.at[sidx], arows, gsem).wait()
            pltpu.async_copy(btab.at[didx], brows, gsem).wait()

            def dot16(r16):
                acc = jnp.zeros((16,), jnp.float32)
                for j in range(ZD):
                    cj = jnp.full((16,), j, jnp.int32)
                    va = plsc.load_gather(arows, [r16, cj])
                    vb = plsc.load_gather(brows, [r16, cj])
                    acc = acc + va * vb
                return acc

            def grp(g, carry2):
                r16 = g * 16 + iota16
                outv[pl.ds(g * 16, 16)] = dot16(r16) + bvec
                return carry2
            lax.fori_loop(0, FULL_GROUPS, grp, 0)

            msk = iota16 < TAIL
            r16t = jnp.where(msk, FULL_GROUPS * 16 + iota16, 0)
            plsc.store_compressed(outv.at[pl.ds(FULL_GROUPS * 16, 16)],
                                  dot16(r16t) + bvec, mask=msk)
            pltpu.sync_copy(outv.at[pl.ds(0, CH)], out_hbm.at[pl.ds(eb, CH)])
            return carry
        lax.fori_loop(0, BIL_CHUNKS, ch_body, 0)

    relation(au, zi_t, srcC, dstC, outC, biasC)
    relation(ai, zu_t, srcR, dstR, outR, biasR)


def _sc_bilinear(au, zi, ai, zu, srcC, dstC, srcR, dstR, bbc, bbr):
    mesh = plsc.VectorSubcoreMesh(core_axis_name="c", subcore_axis_name="s", num_cores=NC, num_subcores=NS)
    biasC = jnp.broadcast_to(bbc, (16,)).astype(jnp.float32)
    biasR = jnp.broadcast_to(bbr, (16,)).astype(jnp.float32)
    f = pl.kernel(
        _bil_body,
        compiler_params=pltpu.CompilerParams(use_tc_tiling_on_sc=False, needs_layout_passes=False),
        out_type=[jax.ShapeDtypeStruct((E,), jnp.float32)] * 2,
        mesh=mesh,
        scratch_types=[
            pltpu.VMEM((CH,), jnp.int32),
            pltpu.VMEM((CH,), jnp.int32),
            pltpu.VMEM((CH, ZD), jnp.float32),
            pltpu.VMEM((CH, ZD), jnp.float32),
            pltpu.VMEM((CH + 8,), jnp.float32),
            pltpu.VMEM((16,), jnp.float32),
            pltpu.SemaphoreType.DMA,
        ],
    )
    return f(au, zi, ai, zu, srcC, dstC, srcR, dstR, biasC, biasR)


def kernel(x_user, x_item, ei_clicks, ei_rev, emb_user, emb_item,
           pre_W_user, pre_b_user, pre_W_item, pre_b_item,
           lin_l_W_clicks, lin_l_b_clicks, lin_r_W_clicks,
           lin_l_W_rev, lin_l_b_rev, lin_r_W_rev,
           proj_W_user, proj_b_user, proj_W_item, proj_b_item,
           head_W_user, head_b_user, head_W_item, head_b_item,
           bil_W_clicks, bil_b_clicks, bil_W_rev, bil_b_rev):
    srcC = ei_clicks[0].astype(jnp.int32)
    dstC = ei_clicks[1].astype(jnp.int32)
    srcR = ei_rev[0].astype(jnp.int32)
    dstR = ei_rev[1].astype(jnp.int32)

    h0ulo, h0uhi = _tc_encode(x_user.astype(jnp.int32), emb_user,
                              pre_W_user, pre_b_user)
    h0ilo, h0ihi = _tc_encode(x_item.astype(jnp.int32), emb_item,
                              pre_W_item, pre_b_item)

    sumc_lo, sumc_hi, sumr_lo, sumr_hi, cntc, cntr = _sc_segments(
        h0ulo, h0uhi, h0ilo, h0ihi, srcC, dstC, srcR, dstR)

    zu, au = _tc_dense(sumr_lo, sumr_hi, cntr, h0ulo, h0uhi,
                       lin_l_W_rev, lin_l_b_rev, lin_r_W_rev,
                       proj_W_user, proj_b_user, bil_W_clicks)
    zi, ai = _tc_dense(sumc_lo, sumc_hi, cntc, h0ilo, h0ihi,
                       lin_l_W_clicks, lin_l_b_clicks, lin_r_W_clicks,
                       proj_W_item, proj_b_item, bil_W_rev)

    sc, sr = _sc_bilinear(au, zi, ai, zu, srcC, dstC, srcR, dstR,
                          bil_b_clicks[0], bil_b_rev[0])

    flu = _tc_head(zu, head_W_user, head_b_user).reshape(N, FDIM, BINS)
    fli = _tc_head(zi, head_W_item, head_b_item).reshape(N, FDIM, BINS)

    return (zu, zi, flu, fli, sc, sr)


# speedup vs baseline: 7.2047x; 1.1759x over previous
import functools

import jax
import jax.numpy as jnp
from jax import lax
from jax.experimental import pallas as pl
from jax.experimental.pallas import tpu as pltpu
from jax.experimental.pallas import tpu_sc as plsc

N = 50000
E = 800000
FDIM = 8
BINS = 64
EMB = 16
HID = 64
ZD = 32
HALF = HID // 2

NC = 2
NS = 16

BN = 2000
CH = 1000
SCH = 400
SEG_PER_SUB = E // NS
SEG_CHUNKS = SEG_PER_SUB // SCH
WCH = 400
WTOT = N // WCH
WROUNDS = (WTOT + NS - 1) // NS
BIL_PER_SUB = E // (NC * NS)
BIL_CHUNKS = BIL_PER_SUB // CH
FULL_GROUPS = CH // 16
TAIL = CH - FULL_GROUPS * 16


def _encode_body(x_ref, emb_ref, preW_ref, preb_ref, lo_ref, hi_ref):
    x = x_ref[...]
    acc = jnp.broadcast_to(preb_ref[...][None, :], (BN, HID))
    iota_b = lax.broadcasted_iota(jnp.int32, (1, BINS), 1)
    for f in range(FDIM):
        tf = jnp.dot(emb_ref[f], preW_ref[pl.ds(f * EMB, EMB), :],
                     preferred_element_type=jnp.float32)
        oh = (x[:, f:f + 1] == iota_b).astype(jnp.float32)
        acc = acc + jnp.dot(oh, tf, preferred_element_type=jnp.float32)
    h = jnp.maximum(acc, 0.0)
    lo_ref[...] = h[:, :HALF]
    hi_ref[...] = h[:, HALF:]


def _tc_encode(x, emb, preW, preb):
    grid = N // BN
    return pl.pallas_call(
        _encode_body,
        grid=(grid,),
        in_specs=[
            pl.BlockSpec((BN, FDIM), lambda i: (i, 0)),
            pl.BlockSpec((FDIM, BINS, EMB), lambda i: (0, 0, 0)),
            pl.BlockSpec((FDIM * EMB, HID), lambda i: (0, 0)),
            pl.BlockSpec((HID,), lambda i: (0,)),
        ],
        out_specs=[
            pl.BlockSpec((BN, HALF), lambda i: (i, 0)),
            pl.BlockSpec((BN, HALF), lambda i: (i, 0)),
        ],
        out_shape=[
            jax.ShapeDtypeStruct((N, HALF), jnp.float32),
            jax.ShapeDtypeStruct((N, HALF), jnp.float32),
        ],
    )(x, emb, preW, preb)


def _seg_body(h0ulo, h0uhi, h0ilo, h0ihi, srcC, dstC, srcR, dstR,
              zeros_h, ones_h,
              sumc_lo, sumc_hi, sumr_lo, sumr_hi, cntc, cntr,
              acc, sidx4, didx4, rows2,
              isem0, isem1, isem2, isem3, gsem0, gsem1, ssem0, ssem1):
    c = lax.axis_index("c")
    s = lax.axis_index("s")
    isem = [isem0, isem1, isem2, isem3]
    gsem = [gsem0, gsem1]
    ssem = [ssem0, ssem1]

    def each_chunk(fn):
        def t_body(t, carry):
            k = s + NS * t

            @pl.when(k < WTOT)
            def _():
                fn(k * WCH)
            return carry
        lax.fori_loop(0, WROUNDS, t_body, 0)

    def zero_acc():
        pltpu.sync_copy(zeros_h, rows2.at[0])
        each_chunk(
            lambda r0: pltpu.sync_copy(rows2.at[0], acc.at[pl.ds(r0, WCH)]))

    def seg_phase(table, src, dst, with_gather):
        base = s * SEG_PER_SUB

        def start_idx(k, slot):
            eb = base + k * SCH
            if with_gather:
                pltpu.async_copy(src.at[pl.ds(eb, SCH)], sidx4.at[slot],
                                 isem[slot])
            pltpu.async_copy(dst.at[pl.ds(eb, SCH)], didx4.at[slot],
                             isem[slot])

        def wait_idx(slot):
            if with_gather:
                pltpu.make_async_copy(src.at[pl.ds(0, SCH)], sidx4.at[slot],
                                      isem[slot]).wait()
            pltpu.make_async_copy(dst.at[pl.ds(0, SCH)], didx4.at[slot],
                                  isem[slot]).wait()

        def start_gather(slot, b):
            pltpu.async_copy(table.at[sidx4.at[slot]], rows2.at[b], gsem[b])

        def wait_gather(slot, b):
            pltpu.make_async_copy(table.at[sidx4.at[slot]], rows2.at[b],
                                  gsem[b]).wait()

        def start_scatter(slot, b):
            src_rows = rows2.at[b] if with_gather else rows2.at[0]
            pltpu.async_copy(src_rows, acc.at[didx4.at[slot]], ssem[b],
                             add=True)

        def wait_scatter(slot, b):
            src_rows = rows2.at[b] if with_gather else rows2.at[0]
            pltpu.make_async_copy(src_rows, acc.at[didx4.at[slot]],
                                  ssem[b]).wait()

        start_idx(0, 0)
        start_idx(1, 1)

        def quad(i, carry):
            for b4 in range(4):
                k = 4 * i + b4

                @pl.when(jnp.logical_and(k >= 2, k <= SEG_CHUNKS + 1))
                def _():
                    wait_scatter((b4 + 2) % 4, b4 % 2)

                @pl.when(k + 2 < SEG_CHUNKS)
                def _():
                    start_idx(k + 2, (b4 + 2) % 4)

                if with_gather:
                    @pl.when(k < SEG_CHUNKS)
                    def _():
                        wait_idx(b4)
                        start_gather(b4, b4 % 2)

                    @pl.when(jnp.logical_and(k >= 1, k <= SEG_CHUNKS))
                    def _():
                        wait_gather((b4 + 3) % 4, (b4 + 1) % 2)
                        start_scatter((b4 + 3) % 4, (b4 + 1) % 2)
                else:
                    @pl.when(k < SEG_CHUNKS)
                    def _():
                        wait_idx(b4)
                        start_scatter(b4, b4 % 2)
            return carry
        lax.fori_loop(0, (SEG_CHUNKS + 2 + 3) // 4 + 1, quad, 0)

    def cnt_phase(dst):
        pltpu.sync_copy(ones_h, rows2.at[0])
        seg_phase(None, None, dst, with_gather=False)

    def writeout(out):
        each_chunk(
            lambda r0: pltpu.sync_copy(acc.at[pl.ds(r0, WCH)],
                                       out.at[pl.ds(r0, WCH)]))

    zero_acc()
    plsc.subcore_barrier()

    @pl.when(c == 0)
    def _():
        seg_phase(h0ulo, srcC, dstC, with_gather=True)

    @pl.when(c == 1)
    def _():
        seg_phase(h0uhi, srcC, dstC, with_gather=True)

    plsc.subcore_barrier()

    @pl.when(c == 0)
    def _():
        writeout(sumc_lo)

    @pl.when(c == 1)
    def _():
        writeout(sumc_hi)

    plsc.subcore_barrier()

    zero_acc()
    plsc.subcore_barrier()

    @pl.when(c == 0)
    def _():
        seg_phase(h0ilo, srcR, dstR, with_gather=True)

    @pl.when(c == 1)
    def _():
        seg_phase(h0ihi, srcR, dstR, with_gather=True)

    plsc.subcore_barrier()

    @pl.when(c == 0)
    def _():
        writeout(sumr_lo)

    @pl.when(c == 1)
    def _():
        writeout(sumr_hi)

    plsc.subcore_barrier()

    zero_acc()
    plsc.subcore_barrier()

    @pl.when(c == 0)
    def _():
        cnt_phase(dstC)

    @pl.when(c == 1)
    def _():
        cnt_phase(dstR)

    plsc.subcore_barrier()

    @pl.when(c == 0)
    def _():
        writeout(cntc)

    @pl.when(c == 1)
    def _():
        writeout(cntr)


def _sc_segments(h0ulo, h0uhi, h0ilo, h0ihi, srcC, dstC, srcR, dstR):
    mesh = plsc.VectorSubcoreMesh(core_axis_name="c", subcore_axis_name="s", num_cores=NC, num_subcores=NS)
    zeros_h = jnp.zeros((SCH, HALF), jnp.float32)
    ones_h = jnp.ones((SCH, HALF), jnp.float32)
    f = pl.kernel(
        _seg_body,
        compiler_params=pltpu.CompilerParams(use_tc_tiling_on_sc=False, needs_layout_passes=False),
        out_type=[jax.ShapeDtypeStruct((N, HALF), jnp.float32)] * 4
        + [jax.ShapeDtypeStruct((N, HALF), jnp.float32)] * 2,
        mesh=mesh,
        scratch_types=[
            pltpu.VMEM_SHARED((N, HALF), jnp.float32),
            pltpu.VMEM((4, SCH), jnp.int32),
            pltpu.VMEM((4, SCH), jnp.int32),
            pltpu.VMEM((2, SCH, HALF), jnp.float32),
        ] + [pltpu.SemaphoreType.DMA] * 8,
    )
    return f(h0ulo, h0uhi, h0ilo, h0ihi, srcC, dstC, srcR, dstR, zeros_h, ones_h)


def _dense_body(slo_ref, shi_ref, cnt_ref, h0lo_ref, h0hi_ref,
                lW_ref, lb_ref, rW_ref, pW_ref, pb_ref, bW_ref,
                z_ref, a_ref):
    inv = 1.0 / jnp.maximum(cnt_ref[:, 0:1], 1.0)
    agg_lo = slo_ref[...] * inv
    agg_hi = shi_ref[...] * inv
    h = (jnp.dot(agg_lo, lW_ref[:HALF, :], preferred_element_type=jnp.float32)
         + jnp.dot(agg_hi, lW_ref[HALF:, :], preferred_element_type=jnp.float32)
         + lb_ref[...][None, :]
         + jnp.dot(h0lo_ref[...], rW_ref[:HALF, :], preferred_element_type=jnp.float32)
         + jnp.dot(h0hi_ref[...], rW_ref[HALF:, :], preferred_element_type=jnp.float32))
    z = jnp.maximum(jnp.dot(h, pW_ref[...], preferred_element_type=jnp.float32)
                    + pb_ref[...][None, :], 0.0)
    z_ref[...] = z
    a_ref[...] = jnp.dot(z, bW_ref[...], preferred_element_type=jnp.float32)


def _tc_dense(slo, shi, cnt, h0lo, h0hi, lW, lb, rW, pW, pb, bW):
    grid = N // BN
    full = lambda shape: pl.BlockSpec(shape, lambda i: tuple(0 for _ in shape))
    blk = lambda w: pl.BlockSpec((BN, w), lambda i: (i, 0))
    return pl.pallas_call(
        _dense_body,
        grid=(grid,),
        in_specs=[
            blk(HALF), blk(HALF), blk(HALF), blk(HALF), blk(HALF),
            full((HID, HID)), full((HID,)), full((HID, HID)),
            full((HID, ZD)), full((ZD,)), full((ZD, ZD)),
        ],
        out_specs=[blk(ZD), blk(ZD)],
        out_shape=[
            jax.ShapeDtypeStruct((N, ZD), jnp.float32),
            jax.ShapeDtypeStruct((N, ZD), jnp.float32),
        ],
    )(slo, shi, cnt, h0lo, h0hi, lW, lb, rW, pW, pb, bW)


def _head_body(z_ref, hW_ref, hb_ref, fl_ref):
    fl_ref[...] = (jnp.dot(z_ref[...], hW_ref[...],
                           preferred_element_type=jnp.float32)
                   + hb_ref[...][None, :])


def _tc_head(z, hW, hb):
    grid = N // BN
    return pl.pallas_call(
        _head_body,
        grid=(grid,),
        in_specs=[
            pl.BlockSpec((BN, ZD), lambda i: (i, 0)),
            pl.BlockSpec((ZD, FDIM * BINS), lambda i: (0, 0)),
            pl.BlockSpec((FDIM * BINS,), lambda i: (0,)),
        ],
        out_specs=pl.BlockSpec((BN, FDIM * BINS), lambda i: (i, 0)),
        out_shape=jax.ShapeDtypeStruct((N, FDIM * BINS), jnp.float32),
    )(z, hW, hb)


def _bil_body(au, zi_t, ai, zu_t, srcC, dstC, srcR, dstR, biasC, biasR,
              outC, outR,
              sidx, didx, arows, brows, outv, bias_v, gsem):
    c = lax.axis_index("c")
    s = lax.axis_index("s")
    wid = s * NC + c
    iota16 = lax.iota(jnp.int32, 16)

    def relation(atab, btab, src, dst, out_hbm, bias_h):
        pltpu.sync_copy(bias_h, bias_v)
        bvec = bias_v[...]

        def ch_body(i, carry):
            eb = wid * BIL_PER_SUB + i * CH
            pltpu.sync_copy(src.at[pl.ds(eb, CH)], sidx)
            pltpu.sync_copy(dst.at[pl.ds(eb, CH)], didx)
            pltpu.async_copy(atab.at[sidx], arows, gsem).wait()
            pltpu.async_copy(btab.at[didx], brows, gsem).wait()

            def dot16(r16):
                acc = jnp.zeros((16,), jnp.float32)
                for j in range(ZD):
                    cj = jnp.full((16,), j, jnp.int32)
                    va = plsc.load_gather(arows, [r16, cj])
                    vb = plsc.load_gather(brows, [r16, cj])
                    acc = acc + va * vb
                return acc

            def grp(g, carry2):
                r16 = g * 16 + iota16
                outv[pl.ds(g * 16, 16)] = dot16(r16) + bvec
                return carry2
            lax.fori_loop(0, FULL_GROUPS, grp, 0)

            msk = iota16 < TAIL
            r16t = jnp.where(msk, FULL_GROUPS * 16 + iota16, 0)
            plsc.store_compressed(outv.at[pl.ds(FULL_GROUPS * 16, 16)],
                                  dot16(r16t) + bvec, mask=msk)
            pltpu.sync_copy(outv.at[pl.ds(0, CH)], out_hbm.at[pl.ds(eb, CH)])
            return carry
        lax.fori_loop(0, BIL_CHUNKS, ch_body, 0)

    relation(au, zi_t, srcC, dstC, outC, biasC)
    relation(ai, zu_t, srcR, dstR, outR, biasR)


def _sc_bilinear(au, zi, ai, zu, srcC, dstC, srcR, dstR, bbc, bbr):
    mesh = plsc.VectorSubcoreMesh(core_axis_name="c", subcore_axis_name="s", num_cores=NC, num_subcores=NS)
    biasC = jnp.broadcast_to(bbc, (16,)).astype(jnp.float32)
    biasR = jnp.broadcast_to(bbr, (16,)).astype(jnp.float32)
    f = pl.kernel(
        _bil_body,
        compiler_params=pltpu.CompilerParams(use_tc_tiling_on_sc=False, needs_layout_passes=False),
        out_type=[jax.ShapeDtypeStruct((E,), jnp.float32)] * 2,
        mesh=mesh,
        scratch_types=[
            pltpu.VMEM((CH,), jnp.int32),
            pltpu.VMEM((CH,), jnp.int32),
            pltpu.VMEM((CH, ZD), jnp.float32),
            pltpu.VMEM((CH, ZD), jnp.float32),
            pltpu.VMEM((CH + 8,), jnp.float32),
            pltpu.VMEM((16,), jnp.float32),
            pltpu.SemaphoreType.DMA,
        ],
    )
    return f(au, zi, ai, zu, srcC, dstC, srcR, dstR, biasC, biasR)


def kernel(x_user, x_item, ei_clicks, ei_rev, emb_user, emb_item,
           pre_W_user, pre_b_user, pre_W_item, pre_b_item,
           lin_l_W_clicks, lin_l_b_clicks, lin_r_W_clicks,
           lin_l_W_rev, lin_l_b_rev, lin_r_W_rev,
           proj_W_user, proj_b_user, proj_W_item, proj_b_item,
           head_W_user, head_b_user, head_W_item, head_b_item,
           bil_W_clicks, bil_b_clicks, bil_W_rev, bil_b_rev):
    srcC = ei_clicks[0].astype(jnp.int32)
    dstC = ei_clicks[1].astype(jnp.int32)
    srcR = ei_rev[0].astype(jnp.int32)
    dstR = ei_rev[1].astype(jnp.int32)

    h0ulo, h0uhi = _tc_encode(x_user.astype(jnp.int32), emb_user,
                              pre_W_user, pre_b_user)
    h0ilo, h0ihi = _tc_encode(x_item.astype(jnp.int32), emb_item,
                              pre_W_item, pre_b_item)

    sumc_lo, sumc_hi, sumr_lo, sumr_hi, cntc, cntr = _sc_segments(
        h0ulo, h0uhi, h0ilo, h0ihi, srcC, dstC, srcR, dstR)

    zu, au = _tc_dense(sumr_lo, sumr_hi, cntr, h0ulo, h0uhi,
                       lin_l_W_rev, lin_l_b_rev, lin_r_W_rev,
                       proj_W_user, proj_b_user, bil_W_clicks)
    zi, ai = _tc_dense(sumc_lo, sumc_hi, cntc, h0ilo, h0ihi,
                       lin_l_W_clicks, lin_l_b_clicks, lin_r_W_clicks,
                       proj_W_item, proj_b_item, bil_W_rev)

    sc, sr = _sc_bilinear(au, zi, ai, zu, srcC, dstC, srcR, dstR,
                          bil_b_clicks[0], bil_b_rev[0])

    flu = _tc_head(zu, head_W_user, head_b_user).reshape(N, FDIM, BINS)
    fli = _tc_head(zi, head_W_item, head_b_item).reshape(N, FDIM, BINS)

    return (zu, zi, flu, fli, sc, sr)


# speedup vs baseline: 7.7125x; 1.0705x over previous
import functools

import jax
import jax.numpy as jnp
from jax import lax
from jax.experimental import pallas as pl
from jax.experimental.pallas import tpu as pltpu
from jax.experimental.pallas import tpu_sc as plsc

N = 50000
E = 800000
FDIM = 8
BINS = 64
EMB = 16
HID = 64
ZD = 32
HALF = HID // 2

NC = 2
NS = 16

BN = 2000
SCH = 400
SEG_PER_SUB = E // NS
SEG_CHUNKS = SEG_PER_SUB // SCH
WCH = 400
WTOT = N // WCH
WROUNDS = (WTOT + NS - 1) // NS
BCH = 200
BIL_PER_SUB = E // (NC * NS)
BIL_CHUNKS = BIL_PER_SUB // BCH
FULL_GROUPS = BCH // 16
TAIL = BCH - FULL_GROUPS * 16


def _encode_body(x_ref, emb_ref, preW_ref, preb_ref, lo_ref, hi_ref):
    x = x_ref[...]
    acc = jnp.broadcast_to(preb_ref[...][None, :], (BN, HID))
    iota_b = lax.broadcasted_iota(jnp.int32, (1, BINS), 1)
    for f in range(FDIM):
        tf = jnp.dot(emb_ref[f], preW_ref[pl.ds(f * EMB, EMB), :],
                     preferred_element_type=jnp.float32)
        oh = (x[:, f:f + 1] == iota_b).astype(jnp.float32)
        acc = acc + jnp.dot(oh, tf, preferred_element_type=jnp.float32)
    h = jnp.maximum(acc, 0.0)
    lo_ref[...] = h[:, :HALF]
    hi_ref[...] = h[:, HALF:]


def _tc_encode(x, emb, preW, preb):
    grid = N // BN
    return pl.pallas_call(
        _encode_body,
        grid=(grid,),
        in_specs=[
            pl.BlockSpec((BN, FDIM), lambda i: (i, 0)),
            pl.BlockSpec((FDIM, BINS, EMB), lambda i: (0, 0, 0)),
            pl.BlockSpec((FDIM * EMB, HID), lambda i: (0, 0)),
            pl.BlockSpec((HID,), lambda i: (0,)),
        ],
        out_specs=[
            pl.BlockSpec((BN, HALF), lambda i: (i, 0)),
            pl.BlockSpec((BN, HALF), lambda i: (i, 0)),
        ],
        out_shape=[
            jax.ShapeDtypeStruct((N, HALF), jnp.float32),
            jax.ShapeDtypeStruct((N, HALF), jnp.float32),
        ],
    )(x, emb, preW, preb)


def _seg_body(h0ulo, h0uhi, h0ilo, h0ihi, srcC, dstC, srcR, dstR,
              zeros_h, ones_h,
              sumc_lo, sumc_hi, sumr_lo, sumr_hi, cntc, cntr,
              acc, sidx4, didx4, rows2,
              isem0, isem1, isem2, isem3, gsem0, gsem1, ssem0, ssem1):
    c = lax.axis_index("c")
    s = lax.axis_index("s")
    isem = [isem0, isem1, isem2, isem3]
    gsem = [gsem0, gsem1]
    ssem = [ssem0, ssem1]

    def each_chunk(fn):
        def t_body(t, carry):
            k = s + NS * t

            @pl.when(k < WTOT)
            def _():
                fn(k * WCH)
            return carry
        lax.fori_loop(0, WROUNDS, t_body, 0)

    def zero_acc():
        pltpu.sync_copy(zeros_h, rows2.at[0])
        each_chunk(
            lambda r0: pltpu.sync_copy(rows2.at[0], acc.at[pl.ds(r0, WCH)]))

    def seg_phase(table, src, dst, with_gather):
        base = s * SEG_PER_SUB

        def start_idx(k, slot):
            eb = base + k * SCH
            if with_gather:
                pltpu.async_copy(src.at[pl.ds(eb, SCH)], sidx4.at[slot],
                                 isem[slot])
            pltpu.async_copy(dst.at[pl.ds(eb, SCH)], didx4.at[slot],
                             isem[slot])

        def wait_idx(slot):
            if with_gather:
                pltpu.make_async_copy(src.at[pl.ds(0, SCH)], sidx4.at[slot],
                                      isem[slot]).wait()
            pltpu.make_async_copy(dst.at[pl.ds(0, SCH)], didx4.at[slot],
                                  isem[slot]).wait()

        def start_gather(slot, b):
            pltpu.async_copy(table.at[sidx4.at[slot]], rows2.at[b], gsem[b])

        def wait_gather(slot, b):
            pltpu.make_async_copy(table.at[sidx4.at[slot]], rows2.at[b],
                                  gsem[b]).wait()

        def start_scatter(slot, b):
            src_rows = rows2.at[b] if with_gather else rows2.at[0]
            pltpu.async_copy(src_rows, acc.at[didx4.at[slot]], ssem[b],
                             add=True)

        def wait_scatter(slot, b):
            src_rows = rows2.at[b] if with_gather else rows2.at[0]
            pltpu.make_async_copy(src_rows, acc.at[didx4.at[slot]],
                                  ssem[b]).wait()

        start_idx(0, 0)
        start_idx(1, 1)

        def quad(i, carry):
            for b4 in range(4):
                k = 4 * i + b4

                @pl.when(jnp.logical_and(k >= 2, k <= SEG_CHUNKS + 1))
                def _():
                    wait_scatter((b4 + 2) % 4, b4 % 2)

                @pl.when(k + 2 < SEG_CHUNKS)
                def _():
                    start_idx(k + 2, (b4 + 2) % 4)

                if with_gather:
                    @pl.when(k < SEG_CHUNKS)
                    def _():
                        wait_idx(b4)
                        start_gather(b4, b4 % 2)

                    @pl.when(jnp.logical_and(k >= 1, k <= SEG_CHUNKS))
                    def _():
                        wait_gather((b4 + 3) % 4, (b4 + 1) % 2)
                        start_scatter((b4 + 3) % 4, (b4 + 1) % 2)
                else:
                    @pl.when(k < SEG_CHUNKS)
                    def _():
                        wait_idx(b4)
                        start_scatter(b4, b4 % 2)
            return carry
        lax.fori_loop(0, (SEG_CHUNKS + 2 + 3) // 4 + 1, quad, 0)

    def cnt_phase(dst):
        pltpu.sync_copy(ones_h, rows2.at[0])
        seg_phase(None, None, dst, with_gather=False)

    def writeout(out):
        each_chunk(
            lambda r0: pltpu.sync_copy(acc.at[pl.ds(r0, WCH)],
                                       out.at[pl.ds(r0, WCH)]))

    zero_acc()
    plsc.subcore_barrier()

    @pl.when(c == 0)
    def _():
        seg_phase(h0ulo, srcC, dstC, with_gather=True)

    @pl.when(c == 1)
    def _():
        seg_phase(h0uhi, srcC, dstC, with_gather=True)

    plsc.subcore_barrier()

    @pl.when(c == 0)
    def _():
        writeout(sumc_lo)

    @pl.when(c == 1)
    def _():
        writeout(sumc_hi)

    plsc.subcore_barrier()

    zero_acc()
    plsc.subcore_barrier()

    @pl.when(c == 0)
    def _():
        seg_phase(h0ilo, srcR, dstR, with_gather=True)

    @pl.when(c == 1)
    def _():
        seg_phase(h0ihi, srcR, dstR, with_gather=True)

    plsc.subcore_barrier()

    @pl.when(c == 0)
    def _():
        writeout(sumr_lo)

    @pl.when(c == 1)
    def _():
        writeout(sumr_hi)

    plsc.subcore_barrier()

    zero_acc()
    plsc.subcore_barrier()

    @pl.when(c == 0)
    def _():
        cnt_phase(dstC)

    @pl.when(c == 1)
    def _():
        cnt_phase(dstR)

    plsc.subcore_barrier()

    @pl.when(c == 0)
    def _():
        writeout(cntc)

    @pl.when(c == 1)
    def _():
        writeout(cntr)


def _sc_segments(h0ulo, h0uhi, h0ilo, h0ihi, srcC, dstC, srcR, dstR):
    mesh = plsc.VectorSubcoreMesh(core_axis_name="c", subcore_axis_name="s", num_cores=NC, num_subcores=NS)
    zeros_h = jnp.zeros((SCH, HALF), jnp.float32)
    ones_h = jnp.ones((SCH, HALF), jnp.float32)
    f = pl.kernel(
        _seg_body,
        compiler_params=pltpu.CompilerParams(use_tc_tiling_on_sc=False, needs_layout_passes=False),
        out_type=[jax.ShapeDtypeStruct((N, HALF), jnp.float32)] * 4
        + [jax.ShapeDtypeStruct((N, HALF), jnp.float32)] * 2,
        mesh=mesh,
        scratch_types=[
            pltpu.VMEM_SHARED((N, HALF), jnp.float32),
            pltpu.VMEM((4, SCH), jnp.int32),
            pltpu.VMEM((4, SCH), jnp.int32),
            pltpu.VMEM((2, SCH, HALF), jnp.float32),
        ] + [pltpu.SemaphoreType.DMA] * 8,
    )
    return f(h0ulo, h0uhi, h0ilo, h0ihi, srcC, dstC, srcR, dstR, zeros_h, ones_h)


def _dense_body(slo_ref, shi_ref, cnt_ref, h0lo_ref, h0hi_ref,
                lW_ref, lb_ref, rW_ref, pW_ref, pb_ref, bW_ref,
                z_ref, a_ref):
    inv = 1.0 / jnp.maximum(cnt_ref[:, 0:1], 1.0)
    agg_lo = slo_ref[...] * inv
    agg_hi = shi_ref[...] * inv
    h = (jnp.dot(agg_lo, lW_ref[:HALF, :], preferred_element_type=jnp.float32)
         + jnp.dot(agg_hi, lW_ref[HALF:, :], preferred_element_type=jnp.float32)
         + lb_ref[...][None, :]
         + jnp.dot(h0lo_ref[...], rW_ref[:HALF, :], preferred_element_type=jnp.float32)
         + jnp.dot(h0hi_ref[...], rW_ref[HALF:, :], preferred_element_type=jnp.float32))
    z = jnp.maximum(jnp.dot(h, pW_ref[...], preferred_element_type=jnp.float32)
                    + pb_ref[...][None, :], 0.0)
    z_ref[...] = z
    a_ref[...] = jnp.dot(z, bW_ref[...], preferred_element_type=jnp.float32)


def _tc_dense(slo, shi, cnt, h0lo, h0hi, lW, lb, rW, pW, pb, bW):
    grid = N // BN
    full = lambda shape: pl.BlockSpec(shape, lambda i: tuple(0 for _ in shape))
    blk = lambda w: pl.BlockSpec((BN, w), lambda i: (i, 0))
    return pl.pallas_call(
        _dense_body,
        grid=(grid,),
        in_specs=[
            blk(HALF), blk(HALF), blk(HALF), blk(HALF), blk(HALF),
            full((HID, HID)), full((HID,)), full((HID, HID)),
            full((HID, ZD)), full((ZD,)), full((ZD, ZD)),
        ],
        out_specs=[blk(ZD), blk(ZD)],
        out_shape=[
            jax.ShapeDtypeStruct((N, ZD), jnp.float32),
            jax.ShapeDtypeStruct((N, ZD), jnp.float32),
        ],
    )(slo, shi, cnt, h0lo, h0hi, lW, lb, rW, pW, pb, bW)


def _head_body(z_ref, hW_ref, hb_ref, fl_ref):
    fl_ref[...] = (jnp.dot(z_ref[...], hW_ref[...],
                           preferred_element_type=jnp.float32)
                   + hb_ref[...][None, :])


def _tc_head(z, hW, hb):
    grid = N // BN
    return pl.pallas_call(
        _head_body,
        grid=(grid,),
        in_specs=[
            pl.BlockSpec((BN, ZD), lambda i: (i, 0)),
            pl.BlockSpec((ZD, FDIM * BINS), lambda i: (0, 0)),
            pl.BlockSpec((FDIM * BINS,), lambda i: (0,)),
        ],
        out_specs=pl.BlockSpec((BN, FDIM * BINS), lambda i: (i, 0)),
        out_shape=jax.ShapeDtypeStruct((N, FDIM * BINS), jnp.float32),
    )(z, hW, hb)


def _bil_body(au, zi_t, ai, zu_t, srcC, dstC, srcR, dstR, biasC, biasR,
              outC, outR,
              sidx4, didx4, arows2, brows2, outv2, bias_v,
              isem0, isem1, isem2, isem3, gsem0, gsem1, osem0, osem1):
    c = lax.axis_index("c")
    s = lax.axis_index("s")
    wid = s * NC + c
    iota16 = lax.iota(jnp.int32, 16)
    isem = [isem0, isem1, isem2, isem3]
    gsem = [gsem0, gsem1]
    osem = [osem0, osem1]

    def relation(atab, btab, src, dst, out_hbm, bias_h):
        pltpu.sync_copy(bias_h, bias_v)
        bvec = bias_v[...]
        base = wid * BIL_PER_SUB

        def start_idx(k, slot):
            eb = base + k * BCH
            pltpu.async_copy(src.at[pl.ds(eb, BCH)], sidx4.at[slot],
                             isem[slot])
            pltpu.async_copy(dst.at[pl.ds(eb, BCH)], didx4.at[slot],
                             isem[slot])

        def wait_idx(slot):
            pltpu.make_async_copy(src.at[pl.ds(0, BCH)], sidx4.at[slot],
                                  isem[slot]).wait()
            pltpu.make_async_copy(dst.at[pl.ds(0, BCH)], didx4.at[slot],
                                  isem[slot]).wait()

        def start_gathers(slot, b):
            pltpu.async_copy(atab.at[sidx4.at[slot]], arows2.at[b], gsem[b])
            pltpu.async_copy(btab.at[didx4.at[slot]], brows2.at[b], gsem[b])

        def wait_gathers(slot, b):
            pltpu.make_async_copy(atab.at[sidx4.at[slot]], arows2.at[b],
                                  gsem[b]).wait()
            pltpu.make_async_copy(btab.at[didx4.at[slot]], brows2.at[b],
                                  gsem[b]).wait()

        def dot16(b, r16):
            acc = jnp.zeros((16,), jnp.float32)
            for j in range(ZD):
                cj = jnp.full((16,), j, jnp.int32)
                va = plsc.load_gather(arows2.at[b], [r16, cj])
                vb = plsc.load_gather(brows2.at[b], [r16, cj])
                acc = acc + va * vb
            return acc

        def compute(k, b):
            def grp(g, carry2):
                r16 = g * 16 + iota16
                outv2.at[b][pl.ds(g * 16, 16)] = dot16(b, r16) + bvec
                return carry2
            lax.fori_loop(0, FULL_GROUPS, grp, 0)
            msk = iota16 < TAIL
            r16t = jnp.where(msk, FULL_GROUPS * 16 + iota16, 0)
            plsc.store_compressed(outv2.at[b].at[pl.ds(FULL_GROUPS * 16, 16)],
                                  dot16(b, r16t) + bvec, mask=msk)

        def start_out(k, b):
            eb = base + k * BCH
            pltpu.async_copy(outv2.at[b].at[pl.ds(0, BCH)],
                             out_hbm.at[pl.ds(eb, BCH)], osem[b])

        def wait_out(b):
            pltpu.make_async_copy(outv2.at[b].at[pl.ds(0, BCH)],
                                  out_hbm.at[pl.ds(0, BCH)], osem[b]).wait()

        start_idx(0, 0)
        start_idx(1, 1)

        def quad(i, carry):
            for b4 in range(4):
                k = 4 * i + b4

                @pl.when(jnp.logical_and(k >= 2, k <= BIL_CHUNKS + 1))
                def _():
                    wait_out(b4 % 2)

                @pl.when(k + 2 < BIL_CHUNKS)
                def _():
                    start_idx(k + 2, (b4 + 2) % 4)

                @pl.when(k < BIL_CHUNKS)
                def _():
                    wait_idx(b4)
                    start_gathers(b4, b4 % 2)

                @pl.when(jnp.logical_and(k >= 1, k <= BIL_CHUNKS))
                def _():
                    wait_gathers((b4 + 3) % 4, (b4 + 1) % 2)
                    compute(k - 1, (b4 + 1) % 2)
                    start_out(k - 1, (b4 + 1) % 2)
            return carry
        lax.fori_loop(0, (BIL_CHUNKS + 2 + 3) // 4 + 1, quad, 0)

    relation(au, zi_t, srcC, dstC, outC, biasC)
    relation(ai, zu_t, srcR, dstR, outR, biasR)


def _sc_bilinear(au, zi, ai, zu, srcC, dstC, srcR, dstR, bbc, bbr):
    mesh = plsc.VectorSubcoreMesh(core_axis_name="c", subcore_axis_name="s", num_cores=NC, num_subcores=NS)
    biasC = jnp.broadcast_to(bbc, (16,)).astype(jnp.float32)
    biasR = jnp.broadcast_to(bbr, (16,)).astype(jnp.float32)
    f = pl.kernel(
        _bil_body,
        compiler_params=pltpu.CompilerParams(use_tc_tiling_on_sc=False, needs_layout_passes=False),
        out_type=[jax.ShapeDtypeStruct((E,), jnp.float32)] * 2,
        mesh=mesh,
        scratch_types=[
            pltpu.VMEM((4, BCH), jnp.int32),
            pltpu.VMEM((4, BCH), jnp.int32),
            pltpu.VMEM((2, BCH, ZD), jnp.float32),
            pltpu.VMEM((2, BCH, ZD), jnp.float32),
            pltpu.VMEM((2, BCH + 8), jnp.float32),
            pltpu.VMEM((16,), jnp.float32),
        ] + [pltpu.SemaphoreType.DMA] * 8,
    )
    return f(au, zi, ai, zu, srcC, dstC, srcR, dstR, biasC, biasR)


def kernel(x_user, x_item, ei_clicks, ei_rev, emb_user, emb_item,
           pre_W_user, pre_b_user, pre_W_item, pre_b_item,
           lin_l_W_clicks, lin_l_b_clicks, lin_r_W_clicks,
           lin_l_W_rev, lin_l_b_rev, lin_r_W_rev,
           proj_W_user, proj_b_user, proj_W_item, proj_b_item,
           head_W_user, head_b_user, head_W_item, head_b_item,
           bil_W_clicks, bil_b_clicks, bil_W_rev, bil_b_rev):
    srcC = ei_clicks[0].astype(jnp.int32)
    dstC = ei_clicks[1].astype(jnp.int32)
    srcR = ei_rev[0].astype(jnp.int32)
    dstR = ei_rev[1].astype(jnp.int32)

    h0ulo, h0uhi = _tc_encode(x_user.astype(jnp.int32), emb_user,
                              pre_W_user, pre_b_user)
    h0ilo, h0ihi = _tc_encode(x_item.astype(jnp.int32), emb_item,
                              pre_W_item, pre_b_item)

    sumc_lo, sumc_hi, sumr_lo, sumr_hi, cntc, cntr = _sc_segments(
        h0ulo, h0uhi, h0ilo, h0ihi, srcC, dstC, srcR, dstR)

    zu, au = _tc_dense(sumr_lo, sumr_hi, cntr, h0ulo, h0uhi,
                       lin_l_W_rev, lin_l_b_rev, lin_r_W_rev,
                       proj_W_user, proj_b_user, bil_W_clicks)
    zi, ai = _tc_dense(sumc_lo, sumc_hi, cntc, h0ilo, h0ihi,
                       lin_l_W_clicks, lin_l_b_clicks, lin_r_W_clicks,
                       proj_W_item, proj_b_item, bil_W_rev)

    sc, sr = _sc_bilinear(au, zi, ai, zu, srcC, dstC, srcR, dstR,
                          bil_b_clicks[0], bil_b_rev[0])

    flu = _tc_head(zu, head_W_user, head_b_user).reshape(N, FDIM, BINS)
    fli = _tc_head(zi, head_W_item, head_b_item).reshape(N, FDIM, BINS)

    return (zu, zi, flu, fli, sc, sr)


# speedup vs baseline: 8.0630x; 1.0454x over previous
import functools

import jax
import jax.numpy as jnp
from jax import lax
from jax.experimental import pallas as pl
from jax.experimental.pallas import tpu as pltpu
from jax.experimental.pallas import tpu_sc as plsc

N = 50000
E = 800000
FDIM = 8
BINS = 64
EMB = 16
HID = 64
ZD = 32
HALF = HID // 2

NC = 2
NS = 16

BN = 2000
SCH = 400
SEG_PER_SUB = E // NS
SEG_CHUNKS = SEG_PER_SUB // SCH
WCH = 400
WTOT = N // WCH
WROUNDS = (WTOT + NS - 1) // NS
BCH = 200
BIL_PER_SUB = E // (NC * NS)
BIL_CHUNKS = BIL_PER_SUB // BCH
FULL_GROUPS = BCH // 16
TAIL = BCH - FULL_GROUPS * 16


def _encode_body(x_ref, emb_ref, preW_ref, preb_ref, lo_ref, hi_ref):
    x = x_ref[...]
    acc = jnp.broadcast_to(preb_ref[...][None, :], (BN, HID))
    iota_b = lax.broadcasted_iota(jnp.int32, (1, BINS), 1)
    for f in range(FDIM):
        tf = jnp.dot(emb_ref[f], preW_ref[pl.ds(f * EMB, EMB), :],
                     preferred_element_type=jnp.float32)
        oh = (x[:, f:f + 1] == iota_b).astype(jnp.float32)
        acc = acc + jnp.dot(oh, tf, preferred_element_type=jnp.float32)
    h = jnp.maximum(acc, 0.0)
    lo_ref[...] = h[:, :HALF]
    hi_ref[...] = h[:, HALF:]


def _tc_encode(x, emb, preW, preb):
    grid = N // BN
    return pl.pallas_call(
        _encode_body,
        grid=(grid,),
        in_specs=[
            pl.BlockSpec((BN, FDIM), lambda i: (i, 0)),
            pl.BlockSpec((FDIM, BINS, EMB), lambda i: (0, 0, 0)),
            pl.BlockSpec((FDIM * EMB, HID), lambda i: (0, 0)),
            pl.BlockSpec((HID,), lambda i: (0,)),
        ],
        out_specs=[
            pl.BlockSpec((BN, HALF), lambda i: (i, 0)),
            pl.BlockSpec((BN, HALF), lambda i: (i, 0)),
        ],
        out_shape=[
            jax.ShapeDtypeStruct((N, HALF), jnp.float32),
            jax.ShapeDtypeStruct((N, HALF), jnp.float32),
        ],
    )(x, emb, preW, preb)


def _seg_body(h0ulo, h0uhi, h0ilo, h0ihi, srcC, dstC, srcR, dstR,
              zeros_h,
              sumc_lo, sumc_hi, sumr_lo, sumr_hi,
              acc, sidx4, didx4, rows2,
              isem0, isem1, isem2, isem3, gsem0, gsem1, ssem0, ssem1):
    c = lax.axis_index("c")
    s = lax.axis_index("s")
    isem = [isem0, isem1, isem2, isem3]
    gsem = [gsem0, gsem1]
    ssem = [ssem0, ssem1]

    def each_chunk(fn):
        def t_body(t, carry):
            k = s + NS * t

            @pl.when(k < WTOT)
            def _():
                fn(k * WCH)
            return carry
        lax.fori_loop(0, WROUNDS, t_body, 0)

    def zero_acc():
        pltpu.sync_copy(zeros_h, rows2.at[0])
        each_chunk(
            lambda r0: pltpu.sync_copy(rows2.at[0], acc.at[pl.ds(r0, WCH)]))

    def seg_phase(table, src, dst, with_gather):
        base = s * SEG_PER_SUB

        def start_idx(k, slot):
            eb = base + k * SCH
            if with_gather:
                pltpu.async_copy(src.at[pl.ds(eb, SCH)], sidx4.at[slot],
                                 isem[slot])
            pltpu.async_copy(dst.at[pl.ds(eb, SCH)], didx4.at[slot],
                             isem[slot])

        def wait_idx(slot):
            if with_gather:
                pltpu.make_async_copy(src.at[pl.ds(0, SCH)], sidx4.at[slot],
                                      isem[slot]).wait()
            pltpu.make_async_copy(dst.at[pl.ds(0, SCH)], didx4.at[slot],
                                  isem[slot]).wait()

        def start_gather(slot, b):
            pltpu.async_copy(table.at[sidx4.at[slot]], rows2.at[b], gsem[b])

        def wait_gather(slot, b):
            pltpu.make_async_copy(table.at[sidx4.at[slot]], rows2.at[b],
                                  gsem[b]).wait()

        def start_scatter(slot, b):
            src_rows = rows2.at[b] if with_gather else rows2.at[0]
            pltpu.async_copy(src_rows, acc.at[didx4.at[slot]], ssem[b],
                             add=True)

        def wait_scatter(slot, b):
            src_rows = rows2.at[b] if with_gather else rows2.at[0]
            pltpu.make_async_copy(src_rows, acc.at[didx4.at[slot]],
                                  ssem[b]).wait()

        start_idx(0, 0)
        start_idx(1, 1)

        def quad(i, carry):
            for b4 in range(4):
                k = 4 * i + b4

                @pl.when(jnp.logical_and(k >= 2, k <= SEG_CHUNKS + 1))
                def _():
                    wait_scatter((b4 + 2) % 4, b4 % 2)

                @pl.when(k + 2 < SEG_CHUNKS)
                def _():
                    start_idx(k + 2, (b4 + 2) % 4)

                if with_gather:
                    @pl.when(k < SEG_CHUNKS)
                    def _():
                        wait_idx(b4)
                        start_gather(b4, b4 % 2)

                    @pl.when(jnp.logical_and(k >= 1, k <= SEG_CHUNKS))
                    def _():
                        wait_gather((b4 + 3) % 4, (b4 + 1) % 2)
                        start_scatter((b4 + 3) % 4, (b4 + 1) % 2)
                else:
                    @pl.when(k < SEG_CHUNKS)
                    def _():
                        wait_idx(b4)
                        start_scatter(b4, b4 % 2)
            return carry
        lax.fori_loop(0, (SEG_CHUNKS + 2 + 3) // 4 + 1, quad, 0)

    def writeout(out):
        each_chunk(
            lambda r0: pltpu.sync_copy(acc.at[pl.ds(r0, WCH)],
                                       out.at[pl.ds(r0, WCH)]))

    zero_acc()
    plsc.subcore_barrier()

    @pl.when(c == 0)
    def _():
        seg_phase(h0ulo, srcC, dstC, with_gather=True)

    @pl.when(c == 1)
    def _():
        seg_phase(h0uhi, srcC, dstC, with_gather=True)

    plsc.subcore_barrier()

    @pl.when(c == 0)
    def _():
        writeout(sumc_lo)

    @pl.when(c == 1)
    def _():
        writeout(sumc_hi)

    plsc.subcore_barrier()

    zero_acc()
    plsc.subcore_barrier()

    @pl.when(c == 0)
    def _():
        seg_phase(h0ilo, srcR, dstR, with_gather=True)

    @pl.when(c == 1)
    def _():
        seg_phase(h0ihi, srcR, dstR, with_gather=True)

    plsc.subcore_barrier()

    @pl.when(c == 0)
    def _():
        writeout(sumr_lo)

    @pl.when(c == 1)
    def _():
        writeout(sumr_hi)

    plsc.subcore_barrier()


def _sc_segments(h0ulo, h0uhi, h0ilo, h0ihi, srcC, dstC, srcR, dstR):
    mesh = plsc.VectorSubcoreMesh(core_axis_name="c", subcore_axis_name="s", num_cores=NC, num_subcores=NS)
    zeros_h = jnp.zeros((SCH, HALF), jnp.float32)
    f = pl.kernel(
        _seg_body,
        compiler_params=pltpu.CompilerParams(use_tc_tiling_on_sc=False, needs_layout_passes=False),
        out_type=[jax.ShapeDtypeStruct((N, HALF), jnp.float32)] * 4,
        mesh=mesh,
        scratch_types=[
            pltpu.VMEM_SHARED((N, HALF), jnp.float32),
            pltpu.VMEM((4, SCH), jnp.int32),
            pltpu.VMEM((4, SCH), jnp.int32),
            pltpu.VMEM((2, SCH, HALF), jnp.float32),
        ] + [pltpu.SemaphoreType.DMA] * 8,
    )
    return f(h0ulo, h0uhi, h0ilo, h0ihi, srcC, dstC, srcR, dstR, zeros_h)


RNG = 3136
NPAD = NS * RNG


def _cnt_body(dstC, dstR, cntc, cntr, hist, res, didx4, slab,
              isem0, isem1, isem2, isem3, rsem):
    c = lax.axis_index("c")
    s = lax.axis_index("s")
    isem = [isem0, isem1, isem2, isem3]
    ones16 = jnp.ones((16,), jnp.float32)

    def do_relation(dst, out):
        def zb(i, carry):
            hist[pl.ds(i * 16, 16)] = jnp.zeros((16,), jnp.float32)
            return carry
        lax.fori_loop(0, NPAD // 16, zb, 0)

        base = s * SEG_PER_SUB

        def start_idx(k, slot):
            pltpu.async_copy(dst.at[pl.ds(base + k * SCH, SCH)],
                             didx4.at[slot], isem[slot])

        def wait_idx(slot):
            pltpu.make_async_copy(dst.at[pl.ds(0, SCH)], didx4.at[slot],
                                  isem[slot]).wait()

        start_idx(0, 0)
        start_idx(1, 1)

        def quad(i, carry):
            for b4 in range(4):
                k = 4 * i + b4

                @pl.when(k + 2 < SEG_CHUNKS)
                def _():
                    start_idx(k + 2, (b4 + 2) % 4)

                @pl.when(k < SEG_CHUNKS)
                def _():
                    wait_idx(b4)

                    def grp(g, carry2):
                        dvec = didx4.at[b4][pl.ds(g * 16, 16)]
                        plsc.addupdate_scatter(hist, [dvec], ones16)
                        return carry2
                    lax.fori_loop(0, SCH // 16, grp, 0)
            return carry
        lax.fori_loop(0, (SEG_CHUNKS + 3) // 4 + 1, quad, 0)

        pltpu.sync_copy(hist, slab.at[s])
        plsc.subcore_barrier()
        r0 = s * RNG
        for q in range(NS):
            pltpu.async_copy(slab.at[q].at[pl.ds(r0, RNG)],
                             hist.at[pl.ds(q * RNG, RNG)], rsem)
        for q in range(NS):
            pltpu.make_async_copy(slab.at[q].at[pl.ds(0, RNG)],
                                  hist.at[pl.ds(0, RNG)], rsem).wait()

        def sg(g, carry):
            acc16 = jnp.zeros((16,), jnp.float32)
            for q in range(NS):
                acc16 = acc16 + hist[pl.ds(q * RNG + g * 16, 16)]
            res[pl.ds(g * 16, 16)] = acc16
            return carry
        lax.fori_loop(0, RNG // 16, sg, 0)
        pltpu.sync_copy(res, out.at[pl.ds(r0, RNG)])
        plsc.subcore_barrier()

    @pl.when(c == 0)
    def _():
        do_relation(dstC, cntc)

    @pl.when(c == 1)
    def _():
        do_relation(dstR, cntr)


def _sc_counts(dstC, dstR):
    mesh = plsc.VectorSubcoreMesh(core_axis_name="c", subcore_axis_name="s", num_cores=NC, num_subcores=NS)
    f = pl.kernel(
        _cnt_body,
        compiler_params=pltpu.CompilerParams(use_tc_tiling_on_sc=False, needs_layout_passes=False),
        out_type=[jax.ShapeDtypeStruct((NPAD,), jnp.float32)] * 2,
        mesh=mesh,
        scratch_types=[
            pltpu.VMEM((NPAD,), jnp.float32),
            pltpu.VMEM((RNG,), jnp.float32),
            pltpu.VMEM((4, SCH), jnp.int32),
            pltpu.VMEM_SHARED((NS, NPAD), jnp.float32),
        ] + [pltpu.SemaphoreType.DMA] * 5,
    )
    cc, cr = f(dstC, dstR)
    return cc[:N].reshape(N, 1), cr[:N].reshape(N, 1)


def _dense_body(slo_ref, shi_ref, cnt_ref, h0lo_ref, h0hi_ref,
                lW_ref, lb_ref, rW_ref, pW_ref, pb_ref, bW_ref,
                z_ref, a_ref):
    inv = 1.0 / jnp.maximum(cnt_ref[...], 1.0)
    agg_lo = slo_ref[...] * inv
    agg_hi = shi_ref[...] * inv
    h = (jnp.dot(agg_lo, lW_ref[:HALF, :], preferred_element_type=jnp.float32)
         + jnp.dot(agg_hi, lW_ref[HALF:, :], preferred_element_type=jnp.float32)
         + lb_ref[...][None, :]
         + jnp.dot(h0lo_ref[...], rW_ref[:HALF, :], preferred_element_type=jnp.float32)
         + jnp.dot(h0hi_ref[...], rW_ref[HALF:, :], preferred_element_type=jnp.float32))
    z = jnp.maximum(jnp.dot(h, pW_ref[...], preferred_element_type=jnp.float32)
                    + pb_ref[...][None, :], 0.0)
    z_ref[...] = z
    a_ref[...] = jnp.dot(z, bW_ref[...], preferred_element_type=jnp.float32)


def _tc_dense(slo, shi, cnt, h0lo, h0hi, lW, lb, rW, pW, pb, bW):
    grid = N // BN
    full = lambda shape: pl.BlockSpec(shape, lambda i: tuple(0 for _ in shape))
    blk = lambda w: pl.BlockSpec((BN, w), lambda i: (i, 0))
    return pl.pallas_call(
        _dense_body,
        grid=(grid,),
        in_specs=[
            blk(HALF), blk(HALF), blk(1), blk(HALF), blk(HALF),
            full((HID, HID)), full((HID,)), full((HID, HID)),
            full((HID, ZD)), full((ZD,)), full((ZD, ZD)),
        ],
        out_specs=[blk(ZD), blk(ZD)],
        out_shape=[
            jax.ShapeDtypeStruct((N, ZD), jnp.float32),
            jax.ShapeDtypeStruct((N, ZD), jnp.float32),
        ],
    )(slo, shi, cnt, h0lo, h0hi, lW, lb, rW, pW, pb, bW)


def _head_body(z_ref, hW_ref, hb_ref, fl_ref):
    fl_ref[...] = (jnp.dot(z_ref[...], hW_ref[...],
                           preferred_element_type=jnp.float32)
                   + hb_ref[...][None, :])


def _tc_head(z, hW, hb):
    grid = N // BN
    return pl.pallas_call(
        _head_body,
        grid=(grid,),
        in_specs=[
            pl.BlockSpec((BN, ZD), lambda i: (i, 0)),
            pl.BlockSpec((ZD, FDIM * BINS), lambda i: (0, 0)),
            pl.BlockSpec((FDIM * BINS,), lambda i: (0,)),
        ],
        out_specs=pl.BlockSpec((BN, FDIM * BINS), lambda i: (i, 0)),
        out_shape=jax.ShapeDtypeStruct((N, FDIM * BINS), jnp.float32),
    )(z, hW, hb)


def _bil_body(au, zi_t, ai, zu_t, srcC, dstC, srcR, dstR, biasC, biasR,
              outC, outR,
              sidx4, didx4, arows2, brows2, outv2, bias_v,
              isem0, isem1, isem2, isem3, gsem0, gsem1, osem0, osem1):
    c = lax.axis_index("c")
    s = lax.axis_index("s")
    wid = s * NC + c
    iota16 = lax.iota(jnp.int32, 16)
    isem = [isem0, isem1, isem2, isem3]
    gsem = [gsem0, gsem1]
    osem = [osem0, osem1]

    def relation(atab, btab, src, dst, out_hbm, bias_h):
        pltpu.sync_copy(bias_h, bias_v)
        bvec = bias_v[...]
        base = wid * BIL_PER_SUB

        def start_idx(k, slot):
            eb = base + k * BCH
            pltpu.async_copy(src.at[pl.ds(eb, BCH)], sidx4.at[slot],
                             isem[slot])
            pltpu.async_copy(dst.at[pl.ds(eb, BCH)], didx4.at[slot],
                             isem[slot])

        def wait_idx(slot):
            pltpu.make_async_copy(src.at[pl.ds(0, BCH)], sidx4.at[slot],
                                  isem[slot]).wait()
            pltpu.make_async_copy(dst.at[pl.ds(0, BCH)], didx4.at[slot],
                                  isem[slot]).wait()

        def start_gathers(slot, b):
            pltpu.async_copy(atab.at[sidx4.at[slot]], arows2.at[b], gsem[b])
            pltpu.async_copy(btab.at[didx4.at[slot]], brows2.at[b], gsem[b])

        def wait_gathers(slot, b):
            pltpu.make_async_copy(atab.at[sidx4.at[slot]], arows2.at[b],
                                  gsem[b]).wait()
            pltpu.make_async_copy(btab.at[didx4.at[slot]], brows2.at[b],
                                  gsem[b]).wait()

        def dot16(b, r16):
            acc = jnp.zeros((16,), jnp.float32)
            for j in range(ZD):
                cj = jnp.full((16,), j, jnp.int32)
                va = plsc.load_gather(arows2.at[b], [r16, cj])
                vb = plsc.load_gather(brows2.at[b], [r16, cj])
                acc = acc + va * vb
            return acc

        def compute(k, b):
            def grp(g, carry2):
                r16 = g * 16 + iota16
                outv2.at[b][pl.ds(g * 16, 16)] = dot16(b, r16) + bvec
                return carry2
            lax.fori_loop(0, FULL_GROUPS, grp, 0)
            msk = iota16 < TAIL
            r16t = jnp.where(msk, FULL_GROUPS * 16 + iota16, 0)
            plsc.store_compressed(outv2.at[b].at[pl.ds(FULL_GROUPS * 16, 16)],
                                  dot16(b, r16t) + bvec, mask=msk)

        def start_out(k, b):
            eb = base + k * BCH
            pltpu.async_copy(outv2.at[b].at[pl.ds(0, BCH)],
                             out_hbm.at[pl.ds(eb, BCH)], osem[b])

        def wait_out(b):
            pltpu.make_async_copy(outv2.at[b].at[pl.ds(0, BCH)],
                                  out_hbm.at[pl.ds(0, BCH)], osem[b]).wait()

        start_idx(0, 0)
        start_idx(1, 1)

        def quad(i, carry):
            for b4 in range(4):
                k = 4 * i + b4

                @pl.when(jnp.logical_and(k >= 2, k <= BIL_CHUNKS + 1))
                def _():
                    wait_out(b4 % 2)

                @pl.when(k + 2 < BIL_CHUNKS)
                def _():
                    start_idx(k + 2, (b4 + 2) % 4)

                @pl.when(k < BIL_CHUNKS)
                def _():
                    wait_idx(b4)
                    start_gathers(b4, b4 % 2)

                @pl.when(jnp.logical_and(k >= 1, k <= BIL_CHUNKS))
                def _():
                    wait_gathers((b4 + 3) % 4, (b4 + 1) % 2)
                    compute(k - 1, (b4 + 1) % 2)
                    start_out(k - 1, (b4 + 1) % 2)
            return carry
        lax.fori_loop(0, (BIL_CHUNKS + 2 + 3) // 4 + 1, quad, 0)

    relation(au, zi_t, srcC, dstC, outC, biasC)
    relation(ai, zu_t, srcR, dstR, outR, biasR)


def _sc_bilinear(au, zi, ai, zu, srcC, dstC, srcR, dstR, bbc, bbr):
    mesh = plsc.VectorSubcoreMesh(core_axis_name="c", subcore_axis_name="s", num_cores=NC, num_subcores=NS)
    biasC = jnp.broadcast_to(bbc, (16,)).astype(jnp.float32)
    biasR = jnp.broadcast_to(bbr, (16,)).astype(jnp.float32)
    f = pl.kernel(
        _bil_body,
        compiler_params=pltpu.CompilerParams(use_tc_tiling_on_sc=False, needs_layout_passes=False),
        out_type=[jax.ShapeDtypeStruct((E,), jnp.float32)] * 2,
        mesh=mesh,
        scratch_types=[
            pltpu.VMEM((4, BCH), jnp.int32),
            pltpu.VMEM((4, BCH), jnp.int32),
            pltpu.VMEM((2, BCH, ZD), jnp.float32),
            pltpu.VMEM((2, BCH, ZD), jnp.float32),
            pltpu.VMEM((2, BCH + 8), jnp.float32),
            pltpu.VMEM((16,), jnp.float32),
        ] + [pltpu.SemaphoreType.DMA] * 8,
    )
    return f(au, zi, ai, zu, srcC, dstC, srcR, dstR, biasC, biasR)


def kernel(x_user, x_item, ei_clicks, ei_rev, emb_user, emb_item,
           pre_W_user, pre_b_user, pre_W_item, pre_b_item,
           lin_l_W_clicks, lin_l_b_clicks, lin_r_W_clicks,
           lin_l_W_rev, lin_l_b_rev, lin_r_W_rev,
           proj_W_user, proj_b_user, proj_W_item, proj_b_item,
           head_W_user, head_b_user, head_W_item, head_b_item,
           bil_W_clicks, bil_b_clicks, bil_W_rev, bil_b_rev):
    srcC = ei_clicks[0].astype(jnp.int32)
    dstC = ei_clicks[1].astype(jnp.int32)
    srcR = ei_rev[0].astype(jnp.int32)
    dstR = ei_rev[1].astype(jnp.int32)

    h0ulo, h0uhi = _tc_encode(x_user.astype(jnp.int32), emb_user,
                              pre_W_user, pre_b_user)
    h0ilo, h0ihi = _tc_encode(x_item.astype(jnp.int32), emb_item,
                              pre_W_item, pre_b_item)

    cntc, cntr = _sc_counts(dstC, dstR)
    sumc_lo, sumc_hi, sumr_lo, sumr_hi = _sc_segments(
        h0ulo, h0uhi, h0ilo, h0ihi, srcC, dstC, srcR, dstR)

    zu, au = _tc_dense(sumr_lo, sumr_hi, cntr, h0ulo, h0uhi,
                       lin_l_W_rev, lin_l_b_rev, lin_r_W_rev,
                       proj_W_user, proj_b_user, bil_W_clicks)
    zi, ai = _tc_dense(sumc_lo, sumc_hi, cntc, h0ilo, h0ihi,
                       lin_l_W_clicks, lin_l_b_clicks, lin_r_W_clicks,
                       proj_W_item, proj_b_item, bil_W_rev)

    sc, sr = _sc_bilinear(au, zi, ai, zu, srcC, dstC, srcR, dstR,
                          bil_b_clicks[0], bil_b_rev[0])

    flu = _tc_head(zu, head_W_user, head_b_user).reshape(N, FDIM, BINS)
    fli = _tc_head(zi, head_W_item, head_b_item).reshape(N, FDIM, BINS)

    return (zu, zi, flu, fli, sc, sr)


# speedup vs baseline: 8.3358x; 1.0338x over previous
import functools

import jax
import jax.numpy as jnp
from jax import lax
from jax.experimental import pallas as pl
from jax.experimental.pallas import tpu as pltpu
from jax.experimental.pallas import tpu_sc as plsc

N = 50000
E = 800000
FDIM = 8
BINS = 64
EMB = 16
HID = 64
ZD = 32
HALF = HID // 2

NC = 2
NS = 16

BN = 2000
SCH = 400
SEG_PER_SUB = E // NS
SEG_CHUNKS = SEG_PER_SUB // SCH
WCH = 400
WTOT = N // WCH
WROUNDS = (WTOT + NS - 1) // NS
BCH = 400
BIL_PER_SUB = E // (NC * NS)
BIL_CHUNKS = BIL_PER_SUB // BCH
FULL_GROUPS = BCH // 16
TAIL = BCH - FULL_GROUPS * 16


def _encode_body(x_ref, emb_ref, preW_ref, preb_ref, lo_ref, hi_ref):
    x = x_ref[...]
    acc = jnp.broadcast_to(preb_ref[...][None, :], (BN, HID))
    iota_b = lax.broadcasted_iota(jnp.int32, (1, BINS), 1)
    for f in range(FDIM):
        tf = jnp.dot(emb_ref[f], preW_ref[pl.ds(f * EMB, EMB), :],
                     preferred_element_type=jnp.float32)
        oh = (x[:, f:f + 1] == iota_b).astype(jnp.float32)
        acc = acc + jnp.dot(oh, tf, preferred_element_type=jnp.float32)
    h = jnp.maximum(acc, 0.0)
    lo_ref[...] = h[:, :HALF]
    hi_ref[...] = h[:, HALF:]


def _tc_encode(x, emb, preW, preb):
    grid = N // BN
    return pl.pallas_call(
        _encode_body,
        grid=(grid,),
        in_specs=[
            pl.BlockSpec((BN, FDIM), lambda i: (i, 0)),
            pl.BlockSpec((FDIM, BINS, EMB), lambda i: (0, 0, 0)),
            pl.BlockSpec((FDIM * EMB, HID), lambda i: (0, 0)),
            pl.BlockSpec((HID,), lambda i: (0,)),
        ],
        out_specs=[
            pl.BlockSpec((BN, HALF), lambda i: (i, 0)),
            pl.BlockSpec((BN, HALF), lambda i: (i, 0)),
        ],
        out_shape=[
            jax.ShapeDtypeStruct((N, HALF), jnp.float32),
            jax.ShapeDtypeStruct((N, HALF), jnp.float32),
        ],
    )(x, emb, preW, preb)


def _seg_body(h0ulo, h0uhi, h0ilo, h0ihi, srcC, dstC, srcR, dstR,
              zeros_h,
              sumc_lo, sumc_hi, sumr_lo, sumr_hi,
              acc, sidx4, didx4, rows2,
              isem0, isem1, isem2, isem3, gsem0, gsem1, ssem0, ssem1):
    c = lax.axis_index("c")
    s = lax.axis_index("s")
    isem = [isem0, isem1, isem2, isem3]
    gsem = [gsem0, gsem1]
    ssem = [ssem0, ssem1]

    def each_chunk(fn):
        def t_body(t, carry):
            k = s + NS * t

            @pl.when(k < WTOT)
            def _():
                fn(k * WCH)
            return carry
        lax.fori_loop(0, WROUNDS, t_body, 0)

    def zero_acc():
        pltpu.sync_copy(zeros_h, rows2.at[0])
        each_chunk(
            lambda r0: pltpu.sync_copy(rows2.at[0], acc.at[pl.ds(r0, WCH)]))

    def seg_phase(table, src, dst, with_gather):
        base = s * SEG_PER_SUB

        def start_idx(k, slot):
            eb = base + k * SCH
            if with_gather:
                pltpu.async_copy(src.at[pl.ds(eb, SCH)], sidx4.at[slot],
                                 isem[slot])
            pltpu.async_copy(dst.at[pl.ds(eb, SCH)], didx4.at[slot],
                             isem[slot])

        def wait_idx(slot):
            if with_gather:
                pltpu.make_async_copy(src.at[pl.ds(0, SCH)], sidx4.at[slot],
                                      isem[slot]).wait()
            pltpu.make_async_copy(dst.at[pl.ds(0, SCH)], didx4.at[slot],
                                  isem[slot]).wait()

        def start_gather(slot, b):
            pltpu.async_copy(table.at[sidx4.at[slot]], rows2.at[b], gsem[b])

        def wait_gather(slot, b):
            pltpu.make_async_copy(table.at[sidx4.at[slot]], rows2.at[b],
                                  gsem[b]).wait()

        def start_scatter(slot, b):
            src_rows = rows2.at[b] if with_gather else rows2.at[0]
            pltpu.async_copy(src_rows, acc.at[didx4.at[slot]], ssem[b],
                             add=True)

        def wait_scatter(slot, b):
            src_rows = rows2.at[b] if with_gather else rows2.at[0]
            pltpu.make_async_copy(src_rows, acc.at[didx4.at[slot]],
                                  ssem[b]).wait()

        start_idx(0, 0)
        start_idx(1, 1)

        def quad(i, carry):
            for b4 in range(4):
                k = 4 * i + b4

                @pl.when(jnp.logical_and(k >= 2, k <= SEG_CHUNKS + 1))
                def _():
                    wait_scatter((b4 + 2) % 4, b4 % 2)

                @pl.when(k + 2 < SEG_CHUNKS)
                def _():
                    start_idx(k + 2, (b4 + 2) % 4)

                if with_gather:
                    @pl.when(k < SEG_CHUNKS)
                    def _():
                        wait_idx(b4)
                        start_gather(b4, b4 % 2)

                    @pl.when(jnp.logical_and(k >= 1, k <= SEG_CHUNKS))
                    def _():
                        wait_gather((b4 + 3) % 4, (b4 + 1) % 2)
                        start_scatter((b4 + 3) % 4, (b4 + 1) % 2)
                else:
                    @pl.when(k < SEG_CHUNKS)
                    def _():
                        wait_idx(b4)
                        start_scatter(b4, b4 % 2)
            return carry
        lax.fori_loop(0, (SEG_CHUNKS + 2 + 3) // 4 + 1, quad, 0)

    def writeout(out):
        each_chunk(
            lambda r0: pltpu.sync_copy(acc.at[pl.ds(r0, WCH)],
                                       out.at[pl.ds(r0, WCH)]))

    zero_acc()
    plsc.subcore_barrier()

    @pl.when(c == 0)
    def _():
        seg_phase(h0ulo, srcC, dstC, with_gather=True)

    @pl.when(c == 1)
    def _():
        seg_phase(h0uhi, srcC, dstC, with_gather=True)

    plsc.subcore_barrier()

    @pl.when(c == 0)
    def _():
        writeout(sumc_lo)

    @pl.when(c == 1)
    def _():
        writeout(sumc_hi)

    plsc.subcore_barrier()

    zero_acc()
    plsc.subcore_barrier()

    @pl.when(c == 0)
    def _():
        seg_phase(h0ilo, srcR, dstR, with_gather=True)

    @pl.when(c == 1)
    def _():
        seg_phase(h0ihi, srcR, dstR, with_gather=True)

    plsc.subcore_barrier()

    @pl.when(c == 0)
    def _():
        writeout(sumr_lo)

    @pl.when(c == 1)
    def _():
        writeout(sumr_hi)

    plsc.subcore_barrier()


def _sc_segments(h0ulo, h0uhi, h0ilo, h0ihi, srcC, dstC, srcR, dstR):
    mesh = plsc.VectorSubcoreMesh(core_axis_name="c", subcore_axis_name="s", num_cores=NC, num_subcores=NS)
    zeros_h = jnp.zeros((SCH, HALF), jnp.float32)
    f = pl.kernel(
        _seg_body,
        compiler_params=pltpu.CompilerParams(use_tc_tiling_on_sc=False, needs_layout_passes=False),
        out_type=[jax.ShapeDtypeStruct((N, HALF), jnp.float32)] * 4,
        mesh=mesh,
        scratch_types=[
            pltpu.VMEM_SHARED((N, HALF), jnp.float32),
            pltpu.VMEM((4, SCH), jnp.int32),
            pltpu.VMEM((4, SCH), jnp.int32),
            pltpu.VMEM((2, SCH, HALF), jnp.float32),
        ] + [pltpu.SemaphoreType.DMA] * 8,
    )
    return f(h0ulo, h0uhi, h0ilo, h0ihi, srcC, dstC, srcR, dstR, zeros_h)


RNG = 3136
NPAD = NS * RNG


def _cnt_body(dstC, dstR, cntc, cntr, hist, res, didx4, slab,
              isem0, isem1, isem2, isem3, rsem):
    c = lax.axis_index("c")
    s = lax.axis_index("s")
    isem = [isem0, isem1, isem2, isem3]
    ones16 = jnp.ones((16,), jnp.float32)

    def do_relation(dst, out):
        def zb(i, carry):
            hist[pl.ds(i * 16, 16)] = jnp.zeros((16,), jnp.float32)
            return carry
        lax.fori_loop(0, NPAD // 16, zb, 0)

        base = s * SEG_PER_SUB

        def start_idx(k, slot):
            pltpu.async_copy(dst.at[pl.ds(base + k * SCH, SCH)],
                             didx4.at[slot], isem[slot])

        def wait_idx(slot):
            pltpu.make_async_copy(dst.at[pl.ds(0, SCH)], didx4.at[slot],
                                  isem[slot]).wait()

        start_idx(0, 0)
        start_idx(1, 1)

        def quad(i, carry):
            for b4 in range(4):
                k = 4 * i + b4

                @pl.when(k + 2 < SEG_CHUNKS)
                def _():
                    start_idx(k + 2, (b4 + 2) % 4)

                @pl.when(k < SEG_CHUNKS)
                def _():
                    wait_idx(b4)

                    def grp(g, carry2):
                        dvec = didx4.at[b4][pl.ds(g * 16, 16)]
                        plsc.addupdate_scatter(hist, [dvec], ones16)
                        return carry2
                    lax.fori_loop(0, SCH // 16, grp, 0)
            return carry
        lax.fori_loop(0, (SEG_CHUNKS + 3) // 4 + 1, quad, 0)

        pltpu.sync_copy(hist, slab.at[s])
        plsc.subcore_barrier()
        r0 = s * RNG
        for q in range(NS):
            pltpu.async_copy(slab.at[q].at[pl.ds(r0, RNG)],
                             hist.at[pl.ds(q * RNG, RNG)], rsem)
        for q in range(NS):
            pltpu.make_async_copy(slab.at[q].at[pl.ds(0, RNG)],
                                  hist.at[pl.ds(0, RNG)], rsem).wait()

        def sg(g, carry):
            acc16 = jnp.zeros((16,), jnp.float32)
            for q in range(NS):
                acc16 = acc16 + hist[pl.ds(q * RNG + g * 16, 16)]
            res[pl.ds(g * 16, 16)] = acc16
            return carry
        lax.fori_loop(0, RNG // 16, sg, 0)
        pltpu.sync_copy(res, out.at[pl.ds(r0, RNG)])
        plsc.subcore_barrier()

    @pl.when(c == 0)
    def _():
        do_relation(dstC, cntc)

    @pl.when(c == 1)
    def _():
        do_relation(dstR, cntr)


def _sc_counts(dstC, dstR):
    mesh = plsc.VectorSubcoreMesh(core_axis_name="c", subcore_axis_name="s", num_cores=NC, num_subcores=NS)
    f = pl.kernel(
        _cnt_body,
        compiler_params=pltpu.CompilerParams(use_tc_tiling_on_sc=False, needs_layout_passes=False),
        out_type=[jax.ShapeDtypeStruct((NPAD,), jnp.float32)] * 2,
        mesh=mesh,
        scratch_types=[
            pltpu.VMEM((NPAD,), jnp.float32),
            pltpu.VMEM((RNG,), jnp.float32),
            pltpu.VMEM((4, SCH), jnp.int32),
            pltpu.VMEM_SHARED((NS, NPAD), jnp.float32),
        ] + [pltpu.SemaphoreType.DMA] * 5,
    )
    cc, cr = f(dstC, dstR)
    return cc[:N].reshape(N, 1), cr[:N].reshape(N, 1)


def _dense_body(slo_ref, shi_ref, cnt_ref, h0lo_ref, h0hi_ref,
                lW_ref, lb_ref, rW_ref, pW_ref, pb_ref, bW_ref,
                z_ref, a_ref):
    inv = 1.0 / jnp.maximum(cnt_ref[...], 1.0)
    agg_lo = slo_ref[...] * inv
    agg_hi = shi_ref[...] * inv
    h = (jnp.dot(agg_lo, lW_ref[:HALF, :], preferred_element_type=jnp.float32)
         + jnp.dot(agg_hi, lW_ref[HALF:, :], preferred_element_type=jnp.float32)
         + lb_ref[...][None, :]
         + jnp.dot(h0lo_ref[...], rW_ref[:HALF, :], preferred_element_type=jnp.float32)
         + jnp.dot(h0hi_ref[...], rW_ref[HALF:, :], preferred_element_type=jnp.float32))
    z = jnp.maximum(jnp.dot(h, pW_ref[...], preferred_element_type=jnp.float32)
                    + pb_ref[...][None, :], 0.0)
    z_ref[...] = z
    a_ref[...] = jnp.dot(z, bW_ref[...], preferred_element_type=jnp.float32)


def _tc_dense(slo, shi, cnt, h0lo, h0hi, lW, lb, rW, pW, pb, bW):
    grid = N // BN
    full = lambda shape: pl.BlockSpec(shape, lambda i: tuple(0 for _ in shape))
    blk = lambda w: pl.BlockSpec((BN, w), lambda i: (i, 0))
    return pl.pallas_call(
        _dense_body,
        grid=(grid,),
        in_specs=[
            blk(HALF), blk(HALF), blk(1), blk(HALF), blk(HALF),
            full((HID, HID)), full((HID,)), full((HID, HID)),
            full((HID, ZD)), full((ZD,)), full((ZD, ZD)),
        ],
        out_specs=[blk(ZD), blk(ZD)],
        out_shape=[
            jax.ShapeDtypeStruct((N, ZD), jnp.float32),
            jax.ShapeDtypeStruct((N, ZD), jnp.float32),
        ],
    )(slo, shi, cnt, h0lo, h0hi, lW, lb, rW, pW, pb, bW)


def _head_body(z_ref, hW_ref, hb_ref, fl_ref):
    fl_ref[...] = (jnp.dot(z_ref[...], hW_ref[...],
                           preferred_element_type=jnp.float32)
                   + hb_ref[...][None, :])


def _tc_head(z, hW, hb):
    grid = N // BN
    return pl.pallas_call(
        _head_body,
        grid=(grid,),
        in_specs=[
            pl.BlockSpec((BN, ZD), lambda i: (i, 0)),
            pl.BlockSpec((ZD, FDIM * BINS), lambda i: (0, 0)),
            pl.BlockSpec((FDIM * BINS,), lambda i: (0,)),
        ],
        out_specs=pl.BlockSpec((BN, FDIM * BINS), lambda i: (i, 0)),
        out_shape=jax.ShapeDtypeStruct((N, FDIM * BINS), jnp.float32),
    )(z, hW, hb)


def _bil_body(au, zi_t, ai, zu_t, srcC, dstC, srcR, dstR, biasC, biasR,
              outC, outR,
              sidx4, didx4, arows2, brows2, outv2, bias_v,
              isem0, isem1, isem2, isem3, gsem0, gsem1, osem0, osem1):
    c = lax.axis_index("c")
    s = lax.axis_index("s")
    wid = s * NC + c
    iota16 = lax.iota(jnp.int32, 16)
    isem = [isem0, isem1, isem2, isem3]
    gsem = [gsem0, gsem1]
    osem = [osem0, osem1]

    def relation(atab, btab, src, dst, out_hbm, bias_h):
        pltpu.sync_copy(bias_h, bias_v)
        bvec = bias_v[...]
        base = wid * BIL_PER_SUB

        def start_idx(k, slot):
            eb = base + k * BCH
            pltpu.async_copy(src.at[pl.ds(eb, BCH)], sidx4.at[slot],
                             isem[slot])
            pltpu.async_copy(dst.at[pl.ds(eb, BCH)], didx4.at[slot],
                             isem[slot])

        def wait_idx(slot):
            pltpu.make_async_copy(src.at[pl.ds(0, BCH)], sidx4.at[slot],
                                  isem[slot]).wait()
            pltpu.make_async_copy(dst.at[pl.ds(0, BCH)], didx4.at[slot],
                                  isem[slot]).wait()

        def start_gathers(slot, b):
            pltpu.async_copy(atab.at[sidx4.at[slot]], arows2.at[b], gsem[b])
            pltpu.async_copy(btab.at[didx4.at[slot]], brows2.at[b], gsem[b])

        def wait_gathers(slot, b):
            pltpu.make_async_copy(atab.at[sidx4.at[slot]], arows2.at[b],
                                  gsem[b]).wait()
            pltpu.make_async_copy(btab.at[didx4.at[slot]], brows2.at[b],
                                  gsem[b]).wait()

        def dot16(b, r16):
            acc = jnp.zeros((16,), jnp.float32)
            for j in range(ZD):
                cj = jnp.full((16,), j, jnp.int32)
                va = plsc.load_gather(arows2.at[b], [r16, cj])
                vb = plsc.load_gather(brows2.at[b], [r16, cj])
                acc = acc + va * vb
            return acc

        def compute(k, b):
            def grp(g, carry2):
                r16 = g * 16 + iota16
                outv2.at[b][pl.ds(g * 16, 16)] = dot16(b, r16) + bvec
                return carry2
            lax.fori_loop(0, FULL_GROUPS, grp, 0)
            if TAIL:
                msk = iota16 < TAIL
                r16t = jnp.where(msk, FULL_GROUPS * 16 + iota16, 0)
                plsc.store_compressed(
                    outv2.at[b].at[pl.ds(FULL_GROUPS * 16, 16)],
                    dot16(b, r16t) + bvec, mask=msk)

        def start_out(k, b):
            eb = base + k * BCH
            pltpu.async_copy(outv2.at[b].at[pl.ds(0, BCH)],
                             out_hbm.at[pl.ds(eb, BCH)], osem[b])

        def wait_out(b):
            pltpu.make_async_copy(outv2.at[b].at[pl.ds(0, BCH)],
                                  out_hbm.at[pl.ds(0, BCH)], osem[b]).wait()

        start_idx(0, 0)
        start_idx(1, 1)

        def quad(i, carry):
            for b4 in range(4):
                k = 4 * i + b4

                @pl.when(jnp.logical_and(k >= 2, k <= BIL_CHUNKS + 1))
                def _():
                    wait_out(b4 % 2)

                @pl.when(k + 2 < BIL_CHUNKS)
                def _():
                    start_idx(k + 2, (b4 + 2) % 4)

                @pl.when(k < BIL_CHUNKS)
                def _():
                    wait_idx(b4)
                    start_gathers(b4, b4 % 2)

                @pl.when(jnp.logical_and(k >= 1, k <= BIL_CHUNKS))
                def _():
                    wait_gathers((b4 + 3) % 4, (b4 + 1) % 2)
                    compute(k - 1, (b4 + 1) % 2)
                    start_out(k - 1, (b4 + 1) % 2)
            return carry
        lax.fori_loop(0, (BIL_CHUNKS + 2 + 3) // 4 + 1, quad, 0)

    relation(au, zi_t, srcC, dstC, outC, biasC)
    relation(ai, zu_t, srcR, dstR, outR, biasR)


def _sc_bilinear(au, zi, ai, zu, srcC, dstC, srcR, dstR, bbc, bbr):
    mesh = plsc.VectorSubcoreMesh(core_axis_name="c", subcore_axis_name="s", num_cores=NC, num_subcores=NS)
    biasC = jnp.broadcast_to(bbc, (16,)).astype(jnp.float32)
    biasR = jnp.broadcast_to(bbr, (16,)).astype(jnp.float32)
    f = pl.kernel(
        _bil_body,
        compiler_params=pltpu.CompilerParams(use_tc_tiling_on_sc=False, needs_layout_passes=False),
        out_type=[jax.ShapeDtypeStruct((E,), jnp.float32)] * 2,
        mesh=mesh,
        scratch_types=[
            pltpu.VMEM((4, BCH), jnp.int32),
            pltpu.VMEM((4, BCH), jnp.int32),
            pltpu.VMEM((2, BCH, ZD), jnp.float32),
            pltpu.VMEM((2, BCH, ZD), jnp.float32),
            pltpu.VMEM((2, BCH + 8), jnp.float32),
            pltpu.VMEM((16,), jnp.float32),
        ] + [pltpu.SemaphoreType.DMA] * 8,
    )
    return f(au, zi, ai, zu, srcC, dstC, srcR, dstR, biasC, biasR)


def kernel(x_user, x_item, ei_clicks, ei_rev, emb_user, emb_item,
           pre_W_user, pre_b_user, pre_W_item, pre_b_item,
           lin_l_W_clicks, lin_l_b_clicks, lin_r_W_clicks,
           lin_l_W_rev, lin_l_b_rev, lin_r_W_rev,
           proj_W_user, proj_b_user, proj_W_item, proj_b_item,
           head_W_user, head_b_user, head_W_item, head_b_item,
           bil_W_clicks, bil_b_clicks, bil_W_rev, bil_b_rev):
    srcC = ei_clicks[0].astype(jnp.int32)
    dstC = ei_clicks[1].astype(jnp.int32)
    srcR = ei_rev[0].astype(jnp.int32)
    dstR = ei_rev[1].astype(jnp.int32)

    h0ulo, h0uhi = _tc_encode(x_user.astype(jnp.int32), emb_user,
                              pre_W_user, pre_b_user)
    h0ilo, h0ihi = _tc_encode(x_item.astype(jnp.int32), emb_item,
                              pre_W_item, pre_b_item)

    cntc, cntr = _sc_counts(dstC, dstR)
    sumc_lo, sumc_hi, sumr_lo, sumr_hi = _sc_segments(
        h0ulo, h0uhi, h0ilo, h0ihi, srcC, dstC, srcR, dstR)

    zu, au = _tc_dense(sumr_lo, sumr_hi, cntr, h0ulo, h0uhi,
                       lin_l_W_rev, lin_l_b_rev, lin_r_W_rev,
                       proj_W_user, proj_b_user, bil_W_clicks)
    zi, ai = _tc_dense(sumc_lo, sumc_hi, cntc, h0ilo, h0ihi,
                       lin_l_W_clicks, lin_l_b_clicks, lin_r_W_clicks,
                       proj_W_item, proj_b_item, bil_W_rev)

    sc, sr = _sc_bilinear(au, zi, ai, zu, srcC, dstC, srcR, dstR,
                          bil_b_clicks[0], bil_b_rev[0])

    flu = _tc_head(zu, head_W_user, head_b_user).reshape(N, FDIM, BINS)
    fli = _tc_head(zi, head_W_item, head_b_item).reshape(N, FDIM, BINS)

    return (zu, zi, flu, fli, sc, sr)
